# baseline (speedup 1.0000x reference)
import jax
import jax.numpy as jnp
import numpy as np
from jax import lax
from jax.experimental import pallas as pl
from jax.experimental.pallas import tpu as pltpu
from jax.experimental.pallas import tpu_sc as plsc

PI = np.pi
X_MIN, X_N = 1e-05, 5000
SIGMA_MIN, SIGMA_MAX, SIGMA_N = 0.003, 2, 5000

N_ROWS = 1_000_000
N_REAL = 4 * N_ROWS
ROWS = 8000
N_FLAT = ROWS * 512
CHUNK = 128
N_CHUNKS = N_FLAT // CHUNK
SUP = 16

NUM_CORES = 2
NUM_SUBCORES = 16
NUM_WORKERS = NUM_CORES * NUM_SUBCORES
CHUNKS_PER_WORKER = N_CHUNKS // NUM_WORKERS
PAIRS = 32
CLAMP = CHUNKS_PER_WORKER - SUP

_XA = np.log(X_MIN)
_XSCALE = np.float32(np.float32(X_N) / np.float32(0 - np.log(X_MIN)))
_SA = np.log(SIGMA_MIN)
_SSCALE = np.float32(
    np.float32(SIGMA_N) / np.float32(np.log(SIGMA_MAX) - np.log(SIGMA_MIN))
)


def _sigma_index_body(s_ref, si_ref):
    sb = s_ref[...]
    sl = jnp.log(sb / PI)
    si = (sl - _SA) * _SSCALE
    si = jnp.round(jnp.clip(si, 0, SIGMA_N)).astype(jnp.int32)
    si_ref[...] = si * (X_N + 1)


def _x_index_body(x_ref, si_ref, idx_ref, ns_ref):
    xb = x_ref[...]
    xw = (xb + PI) % (2 * PI) - PI
    sign = jnp.sign(xw)
    xl = jnp.log(jnp.abs(xw) / PI + 1e-10)
    xi = (xl - _XA) * _XSCALE
    xi = jnp.round(jnp.clip(xi, 0, X_N)).astype(jnp.int32)
    idx_ref[...] = jnp.repeat(si_ref[...], 4, axis=1) + xi
    ns_ref[...] = -sign


def _mul_body(g_ref, ns_ref, o_ref):
    o_ref[...] = g_ref[...] * ns_ref[...]


def _tc_indices(x, sigma):
    blk = 64
    sf = jnp.pad(sigma.reshape(-1), (0, ROWS * 128 - N_ROWS)).reshape(ROWS, 128)
    si2d = pl.pallas_call(
        _sigma_index_body,
        grid=(ROWS // 320,),
        in_specs=[pl.BlockSpec((320, 128), lambda i: (i, 0))],
        out_specs=pl.BlockSpec((320, 128), lambda i: (i, 0)),
        out_shape=jax.ShapeDtypeStruct((ROWS, 128), jnp.int32),
    )(sf)

    xf = jnp.pad(x.reshape(-1), (0, N_FLAT - N_REAL)).reshape(ROWS, 512)
    idx2d, ns2d = pl.pallas_call(
        _x_index_body,
        grid=(ROWS // blk,),
        in_specs=[
            pl.BlockSpec((blk, 512), lambda i: (i, 0)),
            pl.BlockSpec((blk, 128), lambda i: (i, 0)),
        ],
        out_specs=[
            pl.BlockSpec((blk, 512), lambda i: (i, 0)),
            pl.BlockSpec((blk, 512), lambda i: (i, 0)),
        ],
        out_shape=[
            jax.ShapeDtypeStruct((ROWS, 512), jnp.int32),
            jax.ShapeDtypeStruct((ROWS, 512), jnp.float32),
        ],
    )(xf, si2d)
    return idx2d, ns2d


def _sc_body(idx_hbm, tbl_hbm, out_hbm,
             idx_a, idx_b, gth_a, gth_b,
             ssem_a, ssem_b, gsem_a, gsem_b, wsem_a, wsem_b):
    wid = lax.axis_index("s") * NUM_CORES + lax.axis_index("c")
    start = wid * CHUNKS_PER_WORKER

    def stage(buf, sem, chunk0):
        pltpu.async_copy(idx_hbm.at[pl.ds(chunk0 * CHUNK, SUP * CHUNK)], buf, sem)

    def stage_wait(buf, sem):
        pltpu.make_async_copy(idx_hbm.at[pl.ds(0, SUP * CHUNK)], buf, sem).wait()

    def writeback(buf, sem, chunk0):
        pltpu.async_copy(buf, out_hbm.at[pl.ds(chunk0 * CHUNK, SUP * CHUNK)], sem)

    def writeback_wait(buf, sem):
        pltpu.make_async_copy(buf, out_hbm.at[pl.ds(0, SUP * CHUNK)], sem).wait()

    stage(idx_a, ssem_a, start)

    def pair_body(i, carry):
        c0 = start + jnp.minimum(i * (2 * SUP), CLAMP)
        c1 = start + jnp.minimum(i * (2 * SUP) + SUP, CLAMP)
        c2 = start + jnp.minimum(i * (2 * SUP) + 2 * SUP, CLAMP)

        stage_wait(idx_a, ssem_a)

        @pl.when(i > 0)
        def _():
            writeback_wait(gth_a, wsem_a)

        cps_a = [
            pltpu.async_copy(tbl_hbm.at[idx_a.at[pl.ds(j * CHUNK, CHUNK)]],
                             gth_a.at[pl.ds(j * CHUNK, CHUNK)], gsem_a)
            for j in range(SUP)
        ]
        stage(idx_b, ssem_b, c1)
        stage_wait(idx_b, ssem_b)

        @pl.when(i > 0)
        def _():
            writeback_wait(gth_b, wsem_b)

        cps_b = [
            pltpu.async_copy(tbl_hbm.at[idx_b.at[pl.ds(j * CHUNK, CHUNK)]],
                             gth_b.at[pl.ds(j * CHUNK, CHUNK)], gsem_b)
            for j in range(SUP)
        ]
        for cp in cps_a:
            cp.wait()
        writeback(gth_a, wsem_a, c0)

        @pl.when(i < PAIRS - 1)
        def _():
            stage(idx_a, ssem_a, c2)

        for cp in cps_b:
            cp.wait()
        writeback(gth_b, wsem_b, c1)
        return carry

    lax.fori_loop(0, PAIRS, pair_body, 0)
    writeback_wait(gth_a, wsem_a)
    writeback_wait(gth_b, wsem_b)


def _sc_gather(idx2d, tbl_f):
    mesh = plsc.VectorSubcoreMesh(core_axis_name="c", subcore_axis_name="s")
    return pl.kernel(
        _sc_body,
        out_type=jax.ShapeDtypeStruct((N_FLAT,), jnp.float32),
        mesh=mesh,
        scratch_types=[
            pltpu.VMEM((SUP * CHUNK,), jnp.int32),
            pltpu.VMEM((SUP * CHUNK,), jnp.int32),
            pltpu.VMEM((SUP * CHUNK,), jnp.float32),
            pltpu.VMEM((SUP * CHUNK,), jnp.float32),
            pltpu.SemaphoreType.DMA,
            pltpu.SemaphoreType.DMA,
            pltpu.SemaphoreType.DMA,
            pltpu.SemaphoreType.DMA,
            pltpu.SemaphoreType.DMA,
            pltpu.SemaphoreType.DMA,
        ],
    )(idx2d, tbl_f)


def kernel(x, sigma, score_table):
    idx2d, ns2d = _tc_indices(x, sigma)
    tbl_f = score_table.reshape(-1)
    gth = _sc_gather(idx2d.reshape(-1), tbl_f).reshape(ROWS, 512)
    out = pl.pallas_call(
        _mul_body,
        grid=(ROWS // 64,),
        in_specs=[
            pl.BlockSpec((64, 512), lambda i: (i, 0)),
            pl.BlockSpec((64, 512), lambda i: (i, 0)),
        ],
        out_specs=pl.BlockSpec((64, 512), lambda i: (i, 0)),
        out_shape=jax.ShapeDtypeStruct((ROWS, 512), jnp.float32),
    )(gth, ns2d)
    return out.reshape(-1)[:N_REAL].reshape(N_ROWS, 4)


# speedup vs baseline: 1.9002x; 1.9002x over previous
import jax
import jax.numpy as jnp
import numpy as np
from jax import lax
from jax.experimental import pallas as pl
from jax.experimental.pallas import tpu as pltpu
from jax.experimental.pallas import tpu_sc as plsc

PI = np.pi
X_MIN, X_N = 1e-05, 5000
SIGMA_MIN, SIGMA_MAX, SIGMA_N = 0.003, 2, 5000

N_ROWS = 1_000_000
COLP = 1 << 20
N_FLAT = 4 * COLP
CHUNK = 128
N_CHUNKS = N_FLAT // CHUNK
SUP = 16
SUP_ELEMS = SUP * CHUNK

XR_ROWS = N_FLAT // 128
SR_ROWS = COLP // 128
SBLOCKS = SR_ROWS // 256

NUM_CORES = 2
NUM_SUBCORES = 16
NUM_WORKERS = NUM_CORES * NUM_SUBCORES
CHUNKS_PER_WORKER = N_CHUNKS // NUM_WORKERS
PAIRS = CHUNKS_PER_WORKER // SUP // 2

_XA = np.log(X_MIN)
_XSCALE = np.float32(np.float32(X_N) / np.float32(0 - np.log(X_MIN)))
_SA = np.log(SIGMA_MIN)
_SSCALE = np.float32(
    np.float32(SIGMA_N) / np.float32(np.log(SIGMA_MAX) - np.log(SIGMA_MIN))
)


def _sigma_index_body(s_ref, si_ref):
    sb = s_ref[...]
    sl = jnp.log(sb / PI)
    si = (sl - _SA) * _SSCALE
    si = jnp.round(jnp.clip(si, 0, SIGMA_N)).astype(jnp.int32)
    si_ref[...] = si * (X_N + 1)


def _x_index_body(x_ref, si_ref, idx_ref, ns_ref):
    xb = x_ref[...]
    xw = (xb + PI) % (2 * PI) - PI
    sign = jnp.sign(xw)
    xl = jnp.log(jnp.abs(xw) / PI + 1e-10)
    xi = (xl - _XA) * _XSCALE
    xi = jnp.round(jnp.clip(xi, 0, X_N)).astype(jnp.int32)
    idx_ref[...] = si_ref[...] + xi
    ns_ref[...] = -sign


def _tc_indices(x, sigma):
    blk = 256
    sf = jnp.pad(sigma.reshape(-1), (0, COLP - N_ROWS)).reshape(SR_ROWS, 128)
    si2d = pl.pallas_call(
        _sigma_index_body,
        grid=(SR_ROWS // blk,),
        in_specs=[pl.BlockSpec((blk, 128), lambda i: (i, 0))],
        out_specs=pl.BlockSpec((blk, 128), lambda i: (i, 0)),
        out_shape=jax.ShapeDtypeStruct((SR_ROWS, 128), jnp.int32),
    )(sf)

    xt = jnp.pad(x.T, ((0, 0), (0, COLP - N_ROWS))).reshape(XR_ROWS, 128)
    idx2d, ns2d = pl.pallas_call(
        _x_index_body,
        grid=(XR_ROWS // blk,),
        in_specs=[
            pl.BlockSpec((blk, 128), lambda i: (i, 0)),
            pl.BlockSpec((blk, 128), lambda i: (i % SBLOCKS, 0)),
        ],
        out_specs=[
            pl.BlockSpec((blk, 128), lambda i: (i, 0)),
            pl.BlockSpec((blk, 128), lambda i: (i, 0)),
        ],
        out_shape=[
            jax.ShapeDtypeStruct((XR_ROWS, 128), jnp.int32),
            jax.ShapeDtypeStruct((XR_ROWS, 128), jnp.float32),
        ],
    )(xt, si2d)
    return idx2d, ns2d


def _sc_body(idx_hbm, ns_hbm, tbl_hbm, out_hbm,
             idx_a, idx_b, ns_a, ns_b, gth_a, gth_b,
             ssem_a, ssem_b, gsem_a, gsem_b, wsem_a, wsem_b):
    wid = lax.axis_index("s") * NUM_CORES + lax.axis_index("c")
    start = wid * CHUNKS_PER_WORKER

    def stage(ibuf, nbuf, sem, chunk0):
        pltpu.async_copy(idx_hbm.at[pl.ds(chunk0, SUP)], ibuf, sem)
        pltpu.async_copy(ns_hbm.at[pl.ds(chunk0, SUP)], nbuf, sem)

    def stage_wait(ibuf, nbuf, sem):
        pltpu.make_async_copy(idx_hbm.at[pl.ds(0, SUP)], ibuf, sem).wait()
        pltpu.make_async_copy(ns_hbm.at[pl.ds(0, SUP)], nbuf, sem).wait()

    def writeback(buf, sem, chunk0):
        pltpu.async_copy(buf, out_hbm.at[pl.ds(chunk0, SUP)], sem)

    def writeback_wait(buf, sem):
        pltpu.make_async_copy(buf, out_hbm.at[pl.ds(0, SUP)], sem).wait()

    def fire(ibuf, gbuf, sem):
        return [
            pltpu.async_copy(tbl_hbm.at[ibuf.at[j]], gbuf.at[j], sem)
            for j in range(SUP)
        ]

    def mul(gbuf, nbuf):
        def body(r, carry):
            for m in range(CHUNK // 16):
                sl = pl.ds(m * 16, 16)
                gbuf[r, sl] = gbuf[r, sl] * nbuf[r, sl]
            return carry

        lax.fori_loop(0, SUP, body, 0)

    stage(idx_a, ns_a, ssem_a, start)

    def pair_body(i, carry):
        c0 = start + i * (2 * SUP)
        c1 = c0 + SUP
        c2 = c0 + 2 * SUP

        stage_wait(idx_a, ns_a, ssem_a)

        @pl.when(i > 0)
        def _():
            writeback_wait(gth_a, wsem_a)

        cps_a = fire(idx_a, gth_a, gsem_a)
        stage(idx_b, ns_b, ssem_b, c1)
        stage_wait(idx_b, ns_b, ssem_b)

        @pl.when(i > 0)
        def _():
            writeback_wait(gth_b, wsem_b)

        cps_b = fire(idx_b, gth_b, gsem_b)
        for cp in cps_a:
            cp.wait()
        mul(gth_a, ns_a)
        writeback(gth_a, wsem_a, c0)

        @pl.when(i < PAIRS - 1)
        def _():
            stage(idx_a, ns_a, ssem_a, c2)

        for cp in cps_b:
            cp.wait()
        mul(gth_b, ns_b)
        writeback(gth_b, wsem_b, c1)
        return carry

    lax.fori_loop(0, PAIRS, pair_body, 0)
    writeback_wait(gth_a, wsem_a)
    writeback_wait(gth_b, wsem_b)


def _sc_gather(idx2d, ns2d, tbl_f):
    mesh = plsc.VectorSubcoreMesh(core_axis_name="c", subcore_axis_name="s")
    return pl.kernel(
        _sc_body,
        out_type=jax.ShapeDtypeStruct((XR_ROWS, 128), jnp.float32),
        mesh=mesh,
        scratch_types=[
            pltpu.VMEM((SUP, CHUNK), jnp.int32),
            pltpu.VMEM((SUP, CHUNK), jnp.int32),
            pltpu.VMEM((SUP, CHUNK), jnp.float32),
            pltpu.VMEM((SUP, CHUNK), jnp.float32),
            pltpu.VMEM((SUP, CHUNK), jnp.float32),
            pltpu.VMEM((SUP, CHUNK), jnp.float32),
            pltpu.SemaphoreType.DMA,
            pltpu.SemaphoreType.DMA,
            pltpu.SemaphoreType.DMA,
            pltpu.SemaphoreType.DMA,
            pltpu.SemaphoreType.DMA,
            pltpu.SemaphoreType.DMA,
        ],
    )(idx2d, ns2d, tbl_f)


def kernel(x, sigma, score_table):
    idx2d, ns2d = _tc_indices(x, sigma)
    tbl_f = score_table.reshape(-1)
    out2d = _sc_gather(idx2d, ns2d, tbl_f)
    return out2d.reshape(4, COLP)[:, :N_ROWS].T


# speedup vs baseline: 1.9168x; 1.0088x over previous
import jax
import jax.numpy as jnp
import numpy as np
from jax import lax
from jax.experimental import pallas as pl
from jax.experimental.pallas import tpu as pltpu
from jax.experimental.pallas import tpu_sc as plsc

PI = np.pi
X_MIN, X_N = 1e-05, 5000
SIGMA_MIN, SIGMA_MAX, SIGMA_N = 0.003, 2, 5000

N_ROWS = 1_000_000
COLP = 1 << 20
N_FLAT = 4 * COLP
CHUNK = 128
N_CHUNKS = N_FLAT // CHUNK
SUP = 16
SUP_ELEMS = SUP * CHUNK

XR_ROWS = N_FLAT // 128
SR_ROWS = COLP // 128
SBLOCKS = SR_ROWS // 256

NUM_CORES = 2
NUM_SUBCORES = 16
NUM_WORKERS = NUM_CORES * NUM_SUBCORES
CHUNKS_PER_WORKER = N_CHUNKS // NUM_WORKERS
PAIRS = CHUNKS_PER_WORKER // SUP // 2

_XA = np.log(X_MIN)
_XSCALE = np.float32(np.float32(X_N) / np.float32(0 - np.log(X_MIN)))
_SA = np.log(SIGMA_MIN)
_SSCALE = np.float32(
    np.float32(SIGMA_N) / np.float32(np.log(SIGMA_MAX) - np.log(SIGMA_MIN))
)


def _sigma_index_body(s_ref, si_ref):
    sb = s_ref[...]
    sl = jnp.log(sb / PI)
    si = (sl - _SA) * _SSCALE
    si = jnp.round(jnp.clip(si, 0, SIGMA_N)).astype(jnp.int32)
    si_ref[...] = si * (X_N + 1)


def _x_index_body(x_ref, si_ref, idx_ref, ns_ref):
    xb = x_ref[...]
    xw = (xb + PI) % (2 * PI) - PI
    sign = jnp.sign(xw)
    xl = jnp.log(jnp.abs(xw) / PI + 1e-10)
    xi = (xl - _XA) * _XSCALE
    xi = jnp.round(jnp.clip(xi, 0, X_N)).astype(jnp.int32)
    idx_ref[...] = si_ref[...] + xi
    ns_ref[...] = -sign


def _tc_indices(x, sigma):
    blk = 256
    sf = jnp.pad(sigma.reshape(-1), (0, COLP - N_ROWS)).reshape(SR_ROWS, 128)
    si2d = pl.pallas_call(
        _sigma_index_body,
        grid=(SR_ROWS // blk,),
        in_specs=[pl.BlockSpec((blk, 128), lambda i: (i, 0))],
        out_specs=pl.BlockSpec((blk, 128), lambda i: (i, 0)),
        out_shape=jax.ShapeDtypeStruct((SR_ROWS, 128), jnp.int32),
    )(sf)

    xt = jnp.pad(x.T, ((0, 0), (0, COLP - N_ROWS))).reshape(XR_ROWS, 128)
    idx2d, ns2d = pl.pallas_call(
        _x_index_body,
        grid=(XR_ROWS // blk,),
        in_specs=[
            pl.BlockSpec((blk, 128), lambda i: (i, 0)),
            pl.BlockSpec((blk, 128), lambda i: (i % SBLOCKS, 0)),
        ],
        out_specs=[
            pl.BlockSpec((blk, 128), lambda i: (i, 0)),
            pl.BlockSpec((blk, 128), lambda i: (i, 0)),
        ],
        out_shape=[
            jax.ShapeDtypeStruct((XR_ROWS, 128), jnp.int32),
            jax.ShapeDtypeStruct((XR_ROWS, 128), jnp.float32),
        ],
    )(xt, si2d)
    return idx2d, ns2d


def _sc_body(idx_hbm, ns_hbm, tbl_hbm, out_hbm,
             idx_a, idx_b, ns_a, ns_b, gth_a, gth_b,
             ssem_a, ssem_b, gsem_a, gsem_b, wsem_a, wsem_b):
    wid = lax.axis_index("s") * NUM_CORES + lax.axis_index("c")
    start = wid * SUP
    stride = NUM_WORKERS * SUP

    def stage(ibuf, nbuf, sem, chunk0):
        pltpu.async_copy(idx_hbm.at[pl.ds(chunk0, SUP)], ibuf, sem)
        pltpu.async_copy(ns_hbm.at[pl.ds(chunk0, SUP)], nbuf, sem)

    def stage_wait(ibuf, nbuf, sem):
        pltpu.make_async_copy(idx_hbm.at[pl.ds(0, SUP)], ibuf, sem).wait()
        pltpu.make_async_copy(ns_hbm.at[pl.ds(0, SUP)], nbuf, sem).wait()

    def writeback(buf, sem, chunk0):
        pltpu.async_copy(buf, out_hbm.at[pl.ds(chunk0, SUP)], sem)

    def writeback_wait(buf, sem):
        pltpu.make_async_copy(buf, out_hbm.at[pl.ds(0, SUP)], sem).wait()

    def fire(ibuf, gbuf, sem):
        return [
            pltpu.async_copy(tbl_hbm.at[ibuf.at[j]], gbuf.at[j], sem)
            for j in range(SUP)
        ]

    def mul(gbuf, nbuf):
        def body(r, carry):
            for m in range(CHUNK // 16):
                sl = pl.ds(m * 16, 16)
                gbuf[r, sl] = gbuf[r, sl] * nbuf[r, sl]
            return carry

        lax.fori_loop(0, SUP, body, 0)

    stage(idx_a, ns_a, ssem_a, start)

    def pair_body(i, carry):
        c0 = start + (2 * i) * stride
        c1 = c0 + stride
        c2 = c0 + 2 * stride

        stage_wait(idx_a, ns_a, ssem_a)

        @pl.when(i > 0)
        def _():
            writeback_wait(gth_a, wsem_a)

        cps_a = fire(idx_a, gth_a, gsem_a)
        stage(idx_b, ns_b, ssem_b, c1)
        stage_wait(idx_b, ns_b, ssem_b)

        @pl.when(i > 0)
        def _():
            writeback_wait(gth_b, wsem_b)

        cps_b = fire(idx_b, gth_b, gsem_b)
        for cp in cps_a:
            cp.wait()
        mul(gth_a, ns_a)
        writeback(gth_a, wsem_a, c0)

        @pl.when(i < PAIRS - 1)
        def _():
            stage(idx_a, ns_a, ssem_a, c2)

        for cp in cps_b:
            cp.wait()
        mul(gth_b, ns_b)
        writeback(gth_b, wsem_b, c1)
        return carry

    lax.fori_loop(0, PAIRS, pair_body, 0)
    writeback_wait(gth_a, wsem_a)
    writeback_wait(gth_b, wsem_b)


def _sc_gather(idx2d, ns2d, tbl_f):
    mesh = plsc.VectorSubcoreMesh(core_axis_name="c", subcore_axis_name="s")
    return pl.kernel(
        _sc_body,
        out_type=jax.ShapeDtypeStruct((XR_ROWS, 128), jnp.float32),
        mesh=mesh,
        scratch_types=[
            pltpu.VMEM((SUP, CHUNK), jnp.int32),
            pltpu.VMEM((SUP, CHUNK), jnp.int32),
            pltpu.VMEM((SUP, CHUNK), jnp.float32),
            pltpu.VMEM((SUP, CHUNK), jnp.float32),
            pltpu.VMEM((SUP, CHUNK), jnp.float32),
            pltpu.VMEM((SUP, CHUNK), jnp.float32),
            pltpu.SemaphoreType.DMA,
            pltpu.SemaphoreType.DMA,
            pltpu.SemaphoreType.DMA,
            pltpu.SemaphoreType.DMA,
            pltpu.SemaphoreType.DMA,
            pltpu.SemaphoreType.DMA,
        ],
    )(idx2d, ns2d, tbl_f)


def kernel(x, sigma, score_table):
    idx2d, ns2d = _tc_indices(x, sigma)
    tbl_f = score_table.reshape(-1)
    out2d = _sc_gather(idx2d, ns2d, tbl_f)
    return out2d.reshape(4, COLP)[:, :N_ROWS].T


# speedup vs baseline: 1.9228x; 1.0031x over previous
import jax
import jax.numpy as jnp
import numpy as np
from jax import lax
from jax.experimental import pallas as pl
from jax.experimental.pallas import tpu as pltpu
from jax.experimental.pallas import tpu_sc as plsc

PI = np.pi
X_MIN, X_N = 1e-05, 5000
SIGMA_MIN, SIGMA_MAX, SIGMA_N = 0.003, 2, 5000

N_ROWS = 1_000_000
COLP = 1 << 20
N_FLAT = 4 * COLP
CHUNK = 128
N_CHUNKS = N_FLAT // CHUNK
SUP = 32
SUP_ELEMS = SUP * CHUNK

XR_ROWS = N_FLAT // 128
SR_ROWS = COLP // 128
SBLOCKS = SR_ROWS // 256

NUM_CORES = 2
NUM_SUBCORES = 16
NUM_WORKERS = NUM_CORES * NUM_SUBCORES
CHUNKS_PER_WORKER = N_CHUNKS // NUM_WORKERS
PAIRS = CHUNKS_PER_WORKER // SUP // 2

_XA = np.log(X_MIN)
_XSCALE = np.float32(np.float32(X_N) / np.float32(0 - np.log(X_MIN)))
_SA = np.log(SIGMA_MIN)
_SSCALE = np.float32(
    np.float32(SIGMA_N) / np.float32(np.log(SIGMA_MAX) - np.log(SIGMA_MIN))
)


def _sigma_index_body(s_ref, si_ref):
    sb = s_ref[...]
    sl = jnp.log(sb / PI)
    si = (sl - _SA) * _SSCALE
    si = jnp.round(jnp.clip(si, 0, SIGMA_N)).astype(jnp.int32)
    si_ref[...] = si * (X_N + 1)


def _x_index_body(x_ref, si_ref, idx_ref, ns_ref):
    xb = x_ref[...]
    xw = (xb + PI) % (2 * PI) - PI
    sign = jnp.sign(xw)
    xl = jnp.log(jnp.abs(xw) / PI + 1e-10)
    xi = (xl - _XA) * _XSCALE
    xi = jnp.round(jnp.clip(xi, 0, X_N)).astype(jnp.int32)
    idx_ref[...] = si_ref[...] + xi
    ns_ref[...] = -sign


def _tc_indices(x, sigma):
    blk = 256
    sf = jnp.pad(sigma.reshape(-1), (0, COLP - N_ROWS)).reshape(SR_ROWS, 128)
    si2d = pl.pallas_call(
        _sigma_index_body,
        grid=(SR_ROWS // blk,),
        in_specs=[pl.BlockSpec((blk, 128), lambda i: (i, 0))],
        out_specs=pl.BlockSpec((blk, 128), lambda i: (i, 0)),
        out_shape=jax.ShapeDtypeStruct((SR_ROWS, 128), jnp.int32),
    )(sf)

    xt = jnp.pad(x.T, ((0, 0), (0, COLP - N_ROWS))).reshape(XR_ROWS, 128)
    idx2d, ns2d = pl.pallas_call(
        _x_index_body,
        grid=(XR_ROWS // blk,),
        in_specs=[
            pl.BlockSpec((blk, 128), lambda i: (i, 0)),
            pl.BlockSpec((blk, 128), lambda i: (i % SBLOCKS, 0)),
        ],
        out_specs=[
            pl.BlockSpec((blk, 128), lambda i: (i, 0)),
            pl.BlockSpec((blk, 128), lambda i: (i, 0)),
        ],
        out_shape=[
            jax.ShapeDtypeStruct((XR_ROWS, 128), jnp.int32),
            jax.ShapeDtypeStruct((XR_ROWS, 128), jnp.float32),
        ],
    )(xt, si2d)
    return idx2d, ns2d


def _sc_body(idx_hbm, ns_hbm, tbl_hbm, out_hbm,
             idx_a, idx_b, ns_a, ns_b, gth_a, gth_b,
             ssem_a, ssem_b, gsem_a, gsem_b, wsem_a, wsem_b):
    wid = lax.axis_index("s") * NUM_CORES + lax.axis_index("c")
    start = wid * SUP
    stride = NUM_WORKERS * SUP

    def stage(ibuf, nbuf, sem, chunk0):
        pltpu.async_copy(idx_hbm.at[pl.ds(chunk0, SUP)], ibuf, sem)
        pltpu.async_copy(ns_hbm.at[pl.ds(chunk0, SUP)], nbuf, sem)

    def stage_wait(ibuf, nbuf, sem):
        pltpu.make_async_copy(idx_hbm.at[pl.ds(0, SUP)], ibuf, sem).wait()
        pltpu.make_async_copy(ns_hbm.at[pl.ds(0, SUP)], nbuf, sem).wait()

    def writeback(buf, sem, chunk0):
        pltpu.async_copy(buf, out_hbm.at[pl.ds(chunk0, SUP)], sem)

    def writeback_wait(buf, sem):
        pltpu.make_async_copy(buf, out_hbm.at[pl.ds(0, SUP)], sem).wait()

    def fire(ibuf, gbuf, sem):
        return [
            pltpu.async_copy(tbl_hbm.at[ibuf.at[j]], gbuf.at[j], sem)
            for j in range(SUP)
        ]

    def mul(gbuf, nbuf):
        def body(r, carry):
            for m in range(CHUNK // 16):
                sl = pl.ds(m * 16, 16)
                gbuf[r, sl] = gbuf[r, sl] * nbuf[r, sl]
            return carry

        lax.fori_loop(0, SUP, body, 0)

    stage(idx_a, ns_a, ssem_a, start)

    def pair_body(i, carry):
        c0 = start + (2 * i) * stride
        c1 = c0 + stride
        c2 = c0 + 2 * stride

        stage_wait(idx_a, ns_a, ssem_a)

        @pl.when(i > 0)
        def _():
            writeback_wait(gth_a, wsem_a)

        cps_a = fire(idx_a, gth_a, gsem_a)
        stage(idx_b, ns_b, ssem_b, c1)
        stage_wait(idx_b, ns_b, ssem_b)

        @pl.when(i > 0)
        def _():
            writeback_wait(gth_b, wsem_b)

        cps_b = fire(idx_b, gth_b, gsem_b)
        for cp in cps_a:
            cp.wait()
        mul(gth_a, ns_a)
        writeback(gth_a, wsem_a, c0)

        @pl.when(i < PAIRS - 1)
        def _():
            stage(idx_a, ns_a, ssem_a, c2)

        for cp in cps_b:
            cp.wait()
        mul(gth_b, ns_b)
        writeback(gth_b, wsem_b, c1)
        return carry

    lax.fori_loop(0, PAIRS, pair_body, 0)
    writeback_wait(gth_a, wsem_a)
    writeback_wait(gth_b, wsem_b)


def _sc_gather(idx2d, ns2d, tbl_f):
    mesh = plsc.VectorSubcoreMesh(core_axis_name="c", subcore_axis_name="s")
    return pl.kernel(
        _sc_body,
        out_type=jax.ShapeDtypeStruct((XR_ROWS, 128), jnp.float32),
        mesh=mesh,
        scratch_types=[
            pltpu.VMEM((SUP, CHUNK), jnp.int32),
            pltpu.VMEM((SUP, CHUNK), jnp.int32),
            pltpu.VMEM((SUP, CHUNK), jnp.float32),
            pltpu.VMEM((SUP, CHUNK), jnp.float32),
            pltpu.VMEM((SUP, CHUNK), jnp.float32),
            pltpu.VMEM((SUP, CHUNK), jnp.float32),
            pltpu.SemaphoreType.DMA,
            pltpu.SemaphoreType.DMA,
            pltpu.SemaphoreType.DMA,
            pltpu.SemaphoreType.DMA,
            pltpu.SemaphoreType.DMA,
            pltpu.SemaphoreType.DMA,
        ],
    )(idx2d, ns2d, tbl_f)


def kernel(x, sigma, score_table):
    idx2d, ns2d = _tc_indices(x, sigma)
    tbl_f = score_table.reshape(-1)
    out2d = _sc_gather(idx2d, ns2d, tbl_f)
    return out2d.reshape(4, COLP)[:, :N_ROWS].T
